# baseline (speedup 1.0000x reference)
import functools

import jax
import jax.numpy as jnp
from jax import lax
from jax.experimental import pallas as pl
from jax.experimental.pallas import tpu as pltpu
from jax.experimental.pallas import tpu_sc as plsc

SEQ = 200

NC, NS = 2, 16
NW = NC * NS

EMBED = 128
LANES = 16
EWORDS = EMBED // 2
WVECS = EWORDS // LANES
EVECS = EMBED // LANES

NBUF = 4
GRP = 2


def _pool_body(emb_hbm, x_hbm, out_hbm, xs_v, rows_v, acc_v, *sems):
    seq = x_hbm.shape[0]
    bpw = acc_v.shape[0]
    wid = lax.axis_index("s") * NC + lax.axis_index("c")
    pltpu.sync_copy(x_hbm.at[:, pl.ds(wid * bpw, bpw)], xs_v)

    def start(s, buf):
        pltpu.make_async_copy(
            emb_hbm.at[xs_v.at[s]], rows_v.at[buf], sems[buf]).start()

    def wait(s, buf):
        pltpu.make_async_copy(
            emb_hbm.at[xs_v.at[s]], rows_v.at[buf], sems[buf]).wait()

    zero = jnp.zeros((LANES,), jnp.float32)

    def zero_body(b, carry):
        for k in range(EVECS):
            acc_v[b, pl.ds(k * LANES, LANES)] = zero
        return carry

    lax.fori_loop(0, bpw, zero_body, 0)

    for c in range(NBUF):
        start(c, c)

    def accumulate(bufs):
        def b_body(b, carry):
            acc = [acc_v[b, pl.ds(k * LANES, LANES)] for k in range(EVECS)]
            for jj in bufs:
                for k in range(EVECS):
                    acc[k] = acc[k] + rows_v[jj, b, pl.ds(k * LANES, LANES)]
            for k in range(EVECS):
                acc_v[b, pl.ds(k * LANES, LANES)] = acc[k]
            return carry

        lax.fori_loop(0, bpw, b_body, 0)

    def outer_body(p, carry):
        s0 = p * NBUF
        for g in range(NBUF // GRP):
            bufs = tuple(range(g * GRP, (g + 1) * GRP))
            for j in bufs:
                wait(s0 + j, j)
            accumulate(bufs)
            for j in bufs:
                @pl.when(s0 + j + NBUF < seq)
                def _():
                    start(s0 + j + NBUF, j)
        return carry

    lax.fori_loop(0, seq // NBUF, outer_body, 0)
    pltpu.sync_copy(acc_v, out_hbm.at[pl.ds(wid * bpw, bpw)])


def _pool(emb_packed, x, batch):
    bpw = batch // NW
    seq = x.shape[0]
    mesh = plsc.VectorSubcoreMesh(core_axis_name="c", subcore_axis_name="s")
    return pl.kernel(
        _pool_body,
        mesh=mesh,
        out_type=jax.ShapeDtypeStruct((batch, EMBED), jnp.float32),
        scratch_types=[
            pltpu.VMEM((seq, bpw), jnp.int32),
            pltpu.VMEM((NBUF, bpw, EMBED), jnp.float32),
            pltpu.VMEM((bpw, EMBED), jnp.float32),
        ] + [pltpu.SemaphoreType.DMA] * NBUF,
    )(emb_packed, x)


def _mlp_body(m_ref, w1_ref, b1_ref, w2_ref, b2_ref, out_ref, *, bm, on, nt):
    m = m_ref[...].astype(jnp.float32) * (1.0 / SEQ)
    h = (jnp.dot(m, w1_ref[...], preferred_element_type=jnp.float32)
         + b1_ref[...]).astype(jnp.bfloat16)
    mx = jnp.full((bm, 1), -1e30, jnp.float32)
    for t in range(nt):
        sl = pl.ds(t * on, on)
        w2t = w2_ref[:, sl].astype(jnp.bfloat16)
        z = (jnp.dot(h, w2t, preferred_element_type=jnp.float32)
             + b2_ref[:, sl])
        out_ref[:, sl] = z
        mx = jnp.maximum(mx, jnp.max(z, axis=1, keepdims=True))
    s = jnp.zeros((bm, 1), jnp.float32)
    for t in range(nt):
        sl = pl.ds(t * on, on)
        s = s + jnp.sum(jnp.exp(out_ref[:, sl] - mx), axis=1, keepdims=True)
    off = mx + jnp.log(s)
    for t in range(nt):
        sl = pl.ds(t * on, on)
        out_ref[:, sl] = out_ref[:, sl] - off


def _mlp(m, W1, b1r, W2b, b2p, out_cols):
    batch, embed = m.shape
    hidden = W1.shape[1]
    opad = W2b.shape[1]
    bm = 128
    nb = batch // bm
    on = 1280
    nt = opad // on
    return pl.pallas_call(
        functools.partial(_mlp_body, bm=bm, on=on, nt=nt),
        grid=(nb,),
        in_specs=[
            pl.BlockSpec((bm, embed), lambda b: (b, 0)),
            pl.BlockSpec((embed, hidden), lambda b: (0, 0)),
            pl.BlockSpec((1, hidden), lambda b: (0, 0)),
            pl.BlockSpec((hidden, opad), lambda b: (0, 0)),
            pl.BlockSpec((1, opad), lambda b: (0, 0)),
        ],
        out_specs=pl.BlockSpec((bm, opad), lambda b: (b, 0)),
        out_shape=jax.ShapeDtypeStruct((batch, out_cols), jnp.float32),
        compiler_params=pltpu.CompilerParams(
            dimension_semantics=("parallel",)),
    )(m, W1, b1r, W2b, b2p)


def kernel(x, emb, W1, b1, W2, b2):
    seq, batch = x.shape
    vocab, embed = emb.shape
    out_cols = W2.shape[1]

    sums = _pool(emb, x.astype(jnp.int32), batch)
    W1p = W1

    opad = ((out_cols + 1279) // 1280) * 1280
    W2b = jnp.pad(W2, ((0, 0), (0, opad - out_cols)))
    b2p = jnp.pad(b2, (0, opad - out_cols),
                  constant_values=-1e30).reshape(1, -1)
    return _mlp(sums, W1p, b1.reshape(1, -1), W2b, b2p, out_cols)


# speedup vs baseline: 1.0401x; 1.0401x over previous
import functools

import jax
import jax.numpy as jnp
from jax import lax
from jax.experimental import pallas as pl
from jax.experimental.pallas import tpu as pltpu
from jax.experimental.pallas import tpu_sc as plsc

SEQ = 200

NC, NS = 2, 16
NW = NC * NS

EMBED = 128
LANES = 16
EWORDS = EMBED // 2
WVECS = EWORDS // LANES
EVECS = EMBED // LANES

NBUF = 4
GRP = 2


def _pool_body(emb_hbm, x_hbm, out_hbm, xs_v, rows_v, acc_v, *sems):
    seq = x_hbm.shape[0]
    bpw = acc_v.shape[0]
    wid = lax.axis_index("s") * NC + lax.axis_index("c")
    pltpu.sync_copy(x_hbm.at[:, pl.ds(wid * bpw, bpw)], xs_v)

    def start(s, buf):
        pltpu.make_async_copy(
            emb_hbm.at[xs_v.at[s]], rows_v.at[buf], sems[buf]).start()

    def wait(s, buf):
        pltpu.make_async_copy(
            emb_hbm.at[xs_v.at[s]], rows_v.at[buf], sems[buf]).wait()

    zero = jnp.zeros((LANES,), jnp.float32)

    def zero_body(b, carry):
        for k in range(EVECS):
            acc_v[b, pl.ds(k * LANES, LANES)] = zero
        return carry

    lax.fori_loop(0, bpw, zero_body, 0)

    for c in range(NBUF):
        start(c, c)

    def accumulate(bufs):
        def b_body(b, carry):
            acc = [acc_v[b, pl.ds(k * LANES, LANES)] for k in range(EVECS)]
            for jj in bufs:
                for k in range(EVECS):
                    acc[k] = acc[k] + rows_v[jj, b, pl.ds(k * LANES, LANES)]
            for k in range(EVECS):
                acc_v[b, pl.ds(k * LANES, LANES)] = acc[k]
            return carry

        lax.fori_loop(0, bpw, b_body, 0)

    def outer_body(p, carry):
        s0 = p * NBUF
        for g in range(NBUF // GRP):
            bufs = tuple(range(g * GRP, (g + 1) * GRP))
            for j in bufs:
                wait(s0 + j, j)
            accumulate(bufs)
            for j in bufs:
                @pl.when(s0 + j + NBUF < seq)
                def _():
                    start(s0 + j + NBUF, j)
        return carry

    lax.fori_loop(0, seq // NBUF, outer_body, 0)
    pltpu.sync_copy(acc_v, out_hbm.at[pl.ds(wid * bpw, bpw)])


def _pool(emb_packed, x, batch):
    bpw = batch // NW
    seq = x.shape[0]
    mesh = plsc.VectorSubcoreMesh(core_axis_name="c", subcore_axis_name="s")
    return pl.kernel(
        _pool_body,
        mesh=mesh,
        out_type=jax.ShapeDtypeStruct((batch, EMBED), jnp.float32),
        scratch_types=[
            pltpu.VMEM((seq, bpw), jnp.int32),
            pltpu.VMEM((NBUF, bpw, EMBED), jnp.float32),
            pltpu.VMEM((bpw, EMBED), jnp.float32),
        ] + [pltpu.SemaphoreType.DMA] * NBUF,
    )(emb_packed, x)


def _mlp_body(m_ref, w1_ref, b1_ref, w2_ref, b2_ref, out_ref, *, bm, on, nt):
    m = m_ref[...].astype(jnp.float32) * (1.0 / SEQ)
    h = (jnp.dot(m, w1_ref[...], preferred_element_type=jnp.float32)
         + b1_ref[...]).astype(jnp.bfloat16)
    mx = jnp.full((bm, 1), -1e30, jnp.float32)
    for t in range(nt):
        sl = pl.ds(t * on, on)
        z = (jnp.dot(h, w2_ref[:, sl], preferred_element_type=jnp.float32)
             + b2_ref[:, sl])
        out_ref[:, sl] = z
        mx = jnp.maximum(mx, jnp.max(z, axis=1, keepdims=True))
    s = jnp.zeros((bm, 1), jnp.float32)
    for t in range(nt):
        sl = pl.ds(t * on, on)
        s = s + jnp.sum(jnp.exp(out_ref[:, sl] - mx), axis=1, keepdims=True)
    off = mx + jnp.log(s)
    for t in range(nt):
        sl = pl.ds(t * on, on)
        out_ref[:, sl] = out_ref[:, sl] - off


def _mlp(m, W1, b1r, W2b, b2p, out_cols):
    batch, embed = m.shape
    hidden = W1.shape[1]
    opad = W2b.shape[1]
    bm = 256
    nb = batch // bm
    on = 1280
    nt = opad // on
    return pl.pallas_call(
        functools.partial(_mlp_body, bm=bm, on=on, nt=nt),
        grid=(nb,),
        in_specs=[
            pl.BlockSpec((bm, embed), lambda b: (b, 0)),
            pl.BlockSpec((embed, hidden), lambda b: (0, 0)),
            pl.BlockSpec((1, hidden), lambda b: (0, 0)),
            pl.BlockSpec((hidden, opad), lambda b: (0, 0)),
            pl.BlockSpec((1, opad), lambda b: (0, 0)),
        ],
        out_specs=pl.BlockSpec((bm, opad), lambda b: (b, 0)),
        out_shape=jax.ShapeDtypeStruct((batch, out_cols), jnp.float32),
        compiler_params=pltpu.CompilerParams(
            dimension_semantics=("parallel",)),
    )(m, W1, b1r, W2b, b2p)


def kernel(x, emb, W1, b1, W2, b2):
    seq, batch = x.shape
    vocab, embed = emb.shape
    out_cols = W2.shape[1]

    sums = _pool(emb, x.astype(jnp.int32), batch)
    W1p = W1

    opad = ((out_cols + 1279) // 1280) * 1280
    W2b = jnp.pad(W2, ((0, 0), (0, opad - out_cols))).astype(jnp.bfloat16)
    b2p = jnp.pad(b2, (0, opad - out_cols),
                  constant_values=-1e30).reshape(1, -1)
    return _mlp(sums, W1p, b1.reshape(1, -1), W2b, b2p, out_cols)


# speedup vs baseline: 1.0443x; 1.0041x over previous
import functools

import jax
import jax.numpy as jnp
from jax import lax
from jax.experimental import pallas as pl
from jax.experimental.pallas import tpu as pltpu
from jax.experimental.pallas import tpu_sc as plsc

SEQ = 200

NC, NS = 2, 16
NW = NC * NS

EMBED = 128
LANES = 16
EWORDS = EMBED // 2
WVECS = EWORDS // LANES
EVECS = EMBED // LANES

NBUF = 8
GRP = 4


def _pool_body(emb_hbm, x_hbm, out_hbm, xs_v, rows_v, acc_v, *sems):
    seq = x_hbm.shape[0]
    bpw = acc_v.shape[0]
    wid = lax.axis_index("s") * NC + lax.axis_index("c")
    pltpu.sync_copy(x_hbm.at[:, pl.ds(wid * bpw, bpw)], xs_v)

    def start(s, buf):
        pltpu.make_async_copy(
            emb_hbm.at[xs_v.at[s]], rows_v.at[buf], sems[buf]).start()

    def wait(s, buf):
        pltpu.make_async_copy(
            emb_hbm.at[xs_v.at[s]], rows_v.at[buf], sems[buf]).wait()

    zero = jnp.zeros((LANES,), jnp.float32)

    def zero_body(b, carry):
        for k in range(EVECS):
            acc_v[b, pl.ds(k * LANES, LANES)] = zero
        return carry

    lax.fori_loop(0, bpw, zero_body, 0)

    for c in range(NBUF):
        start(c, c)

    def accumulate(bufs):
        def b_body(b, carry):
            acc = [acc_v[b, pl.ds(k * LANES, LANES)] for k in range(EVECS)]
            for jj in bufs:
                for k in range(WVECS):
                    w = rows_v[jj, b, pl.ds(k * LANES, LANES)]
                    acc[k] = acc[k] + lax.bitcast_convert_type(
                        w << 16, jnp.float32)
                    acc[WVECS + k] = acc[WVECS + k] + lax.bitcast_convert_type(
                        w, jnp.float32)
            for k in range(EVECS):
                acc_v[b, pl.ds(k * LANES, LANES)] = acc[k]
            return carry

        lax.fori_loop(0, bpw, b_body, 0)

    def outer_body(p, carry):
        s0 = p * NBUF
        for g in range(NBUF // GRP):
            bufs = tuple(range(g * GRP, (g + 1) * GRP))
            for j in bufs:
                wait(s0 + j, j)
            accumulate(bufs)
            for j in bufs:
                @pl.when(s0 + j + NBUF < seq)
                def _():
                    start(s0 + j + NBUF, j)
        return carry

    lax.fori_loop(0, seq // NBUF, outer_body, 0)
    pltpu.sync_copy(acc_v, out_hbm.at[pl.ds(wid * bpw, bpw)])


def _pool(emb_packed, x, batch):
    bpw = batch // NW
    seq = x.shape[0]
    mesh = plsc.VectorSubcoreMesh(core_axis_name="c", subcore_axis_name="s")
    return pl.kernel(
        _pool_body,
        mesh=mesh,
        compiler_params=pltpu.CompilerParams(use_tc_tiling_on_sc=False),
        out_type=jax.ShapeDtypeStruct((batch, EMBED), jnp.float32),
        scratch_types=[
            pltpu.VMEM((seq, bpw), jnp.int32),
            pltpu.VMEM((NBUF, bpw, EWORDS), jnp.int32),
            pltpu.VMEM((bpw, EMBED), jnp.float32),
        ] + [pltpu.SemaphoreType.DMA] * NBUF,
    )(emb_packed, x)


def _pack_body(emb_ref, out_ref):
    r = lax.bitcast_convert_type(emb_ref[...], jnp.int32)
    rb = (r + 0x7FFF + ((r >> 16) & 1)) >> 16
    lo = rb[:, :EWORDS] & 0xFFFF
    hi = rb[:, EWORDS:] << 16
    out_ref[...] = lo | hi


def _pack(emb):
    vocab = emb.shape[0]
    vb = 4000
    return pl.pallas_call(
        _pack_body,
        grid=(vocab // vb,),
        in_specs=[pl.BlockSpec((vb, EMBED), lambda v: (v, 0))],
        out_specs=pl.BlockSpec((vb, EWORDS), lambda v: (v, 0)),
        out_shape=jax.ShapeDtypeStruct((vocab, EWORDS), jnp.int32),
        compiler_params=pltpu.CompilerParams(
            dimension_semantics=("parallel",)),
    )(emb)


def _mlp_body(m_ref, w1_ref, b1_ref, w2_ref, b2_ref, out_ref, *, bm, on, nt):
    m = m_ref[...].astype(jnp.float32) * (1.0 / SEQ)
    h = (jnp.dot(m, w1_ref[...], preferred_element_type=jnp.float32)
         + b1_ref[...]).astype(jnp.bfloat16)
    mx = jnp.full((bm, 1), -1e30, jnp.float32)
    for t in range(nt):
        sl = pl.ds(t * on, on)
        z = (jnp.dot(h, w2_ref[:, sl], preferred_element_type=jnp.float32)
             + b2_ref[:, sl])
        out_ref[:, sl] = z
        mx = jnp.maximum(mx, jnp.max(z, axis=1, keepdims=True))
    s = jnp.zeros((bm, 1), jnp.float32)
    for t in range(nt):
        sl = pl.ds(t * on, on)
        s = s + jnp.sum(jnp.exp(out_ref[:, sl] - mx), axis=1, keepdims=True)
    off = mx + jnp.log(s)
    for t in range(nt):
        sl = pl.ds(t * on, on)
        out_ref[:, sl] = out_ref[:, sl] - off


def _mlp(m, W1, b1r, W2b, b2p, out_cols):
    batch, embed = m.shape
    hidden = W1.shape[1]
    opad = W2b.shape[1]
    bm = 256
    nb = batch // bm
    on = 1280
    nt = opad // on
    return pl.pallas_call(
        functools.partial(_mlp_body, bm=bm, on=on, nt=nt),
        grid=(nb,),
        in_specs=[
            pl.BlockSpec((bm, embed), lambda b: (b, 0)),
            pl.BlockSpec((embed, hidden), lambda b: (0, 0)),
            pl.BlockSpec((1, hidden), lambda b: (0, 0)),
            pl.BlockSpec((hidden, opad), lambda b: (0, 0)),
            pl.BlockSpec((1, opad), lambda b: (0, 0)),
        ],
        out_specs=pl.BlockSpec((bm, opad), lambda b: (b, 0)),
        out_shape=jax.ShapeDtypeStruct((batch, out_cols), jnp.float32),
        compiler_params=pltpu.CompilerParams(
            dimension_semantics=("parallel",)),
    )(m, W1, b1r, W2b, b2p)


def kernel(x, emb, W1, b1, W2, b2):
    seq, batch = x.shape
    vocab, embed = emb.shape
    out_cols = W2.shape[1]

    sums = _pool(_pack(emb), x.astype(jnp.int32), batch)
    W1p = W1

    opad = ((out_cols + 1279) // 1280) * 1280
    W2b = jnp.pad(W2, ((0, 0), (0, opad - out_cols))).astype(jnp.bfloat16)
    b2p = jnp.pad(b2, (0, opad - out_cols),
                  constant_values=-1e30).reshape(1, -1)
    return _mlp(sums, W1p, b1.reshape(1, -1), W2b, b2p, out_cols)
